# baseline (speedup 1.0000x reference)
import functools

import jax
import jax.numpy as jnp
from jax import lax
from jax.experimental import pallas as pl
from jax.experimental.pallas import tpu as pltpu
from jax.experimental.pallas import tpu_sc as plsc

N_ROWS = 16384
V = 1000000
D = 64
P = 3
NC = 2
NS = 16
NW = NC * NS
BW = 528
B_TOTAL = BW * NW
GW = D * BW
TERM_SLOT = 16


@functools.cache
def _sc_gather_kernel():
    mesh = plsc.VectorSubcoreMesh(core_axis_name="c", subcore_axis_name="s")

    @functools.partial(
        pl.kernel,
        mesh=mesh,
        out_type=jax.ShapeDtypeStruct((NW, GW), jnp.float32),
        scratch_types=[
            pltpu.VMEM((BW,), jnp.int32),
            pltpu.VMEM((GW,), jnp.int32),
            pltpu.VMEM((GW,), jnp.float32),
            pltpu.SemaphoreType.DMA,
        ],
    )
    def _sc_gather(flat_hbm, idx_hbm, out_hbm, idx_v, addr_v, g_v, sem):
        wid = lax.axis_index("s") * NC + lax.axis_index("c")
        base = pl.multiple_of(wid * BW, 8)
        pltpu.sync_copy(idx_hbm.at[pl.ds(base, BW)], idx_v)

        def build(d, carry):
            doff = pl.multiple_of(d * BW, 16)
            for q in range(BW // 16):
                addr_v[pl.ds(doff + 16 * q, 16)] = (
                    idx_v[pl.ds(16 * q, 16)] + d * V
                )
            return carry

        lax.fori_loop(0, D, build, 0)
        pltpu.async_copy(flat_hbm.at[addr_v], g_v, sem).wait()
        pltpu.sync_copy(g_v, out_hbm.at[wid])

    return _sc_gather


def _tc_body(gw_ref, gt_ref, fcw_ref, fcb_ref, fc2t_ref, fc2b_ref, out_ref):
    hi = lax.Precision.HIGHEST
    dims = (((0,), (0,)), ((), ()))
    tr = fcb_ref[...]
    for p in range(P):
        te_p = gt_ref[0, :, TERM_SLOT + p:TERM_SLOT + p + 1]
        tr = tr + lax.dot_general(te_p, fcw_ref[p], dims,
                                  preferred_element_type=jnp.float32,
                                  precision=hi)
    c = jnp.dot(tr, fc2t_ref[...],
                preferred_element_type=jnp.float32, precision=hi) + fc2b_ref[...]
    out_ref[...] = c - lax.dot_general(gw_ref[0], fc2t_ref[...], dims,
                                       preferred_element_type=jnp.float32,
                                       precision=hi)


_tc_call = pl.pallas_call(
    _tc_body,
    grid=(NW,),
    in_specs=[
        pl.BlockSpec((1, D, BW), lambda i: (i, 0, 0)),
        pl.BlockSpec((1, D, BW), lambda i: (NW - 1, 0, 0)),
        pl.BlockSpec((P, D, D), lambda i: (0, 0, 0)),
        pl.BlockSpec((1, D), lambda i: (0, 0)),
        pl.BlockSpec((D, P + 1), lambda i: (0, 0)),
        pl.BlockSpec((1, P + 1), lambda i: (0, 0)),
    ],
    out_specs=pl.BlockSpec((BW, P + 1), lambda i: (i, 0)),
    out_shape=jax.ShapeDtypeStruct((B_TOTAL, P + 1), jnp.float32),
)


def kernel(term, attach_terms, device, table, fc_W, fc_b, fc2_W, fc2_b):
    del device
    idx = jnp.concatenate([
        attach_terms.astype(jnp.int32),
        term.astype(jnp.int32),
        jnp.zeros((B_TOTAL - N_ROWS - P,), jnp.int32),
    ])
    flat_cm = table.T.reshape(-1)
    g = _sc_gather_kernel()(flat_cm, idx)
    g3 = g.reshape(NW, D, BW)
    fc_Wr = fc_W.reshape(D, P, D).transpose(1, 2, 0)
    out = _tc_call(g3, g3, fc_Wr, fc_b.reshape(1, D), fc2_W.T,
                   fc2_b.reshape(1, P + 1))
    return out[:N_ROWS]


# speedup vs baseline: 7.8152x; 7.8152x over previous
import functools

import jax
import jax.numpy as jnp
from jax import lax
from jax.experimental import pallas as pl
from jax.experimental.pallas import tpu as pltpu
from jax.experimental.pallas import tpu_sc as plsc

N_ROWS = 16384
D = 64
P = 3
NC = 2
NS = 16
NW = NC * NS
CHUNK = 104
CHUNKS_PER_W = 5
B_PER_W = CHUNK * CHUNKS_PER_W
B_TOTAL = B_PER_W * NW
NPAIR = 500000


@functools.cache
def _sc_gather_kernel():
    mesh = plsc.VectorSubcoreMesh(core_axis_name="c", subcore_axis_name="s")

    @functools.partial(
        pl.kernel,
        mesh=mesh,
        out_type=jax.ShapeDtypeStruct((B_TOTAL, 2 * D), jnp.float32),
        scratch_types=[
            pltpu.VMEM((B_PER_W,), jnp.int32),
            pltpu.VMEM((B_PER_W, 2 * D), jnp.float32),
            pltpu.SemaphoreType.DMA,
        ],
    )
    def _sc_gather(table_hbm, idx_hbm, out_hbm, idx_v, rows_v, sem):
        wid = lax.axis_index("s") * NC + lax.axis_index("c")
        base = pl.multiple_of(wid * B_PER_W, 8)
        pltpu.sync_copy(idx_hbm.at[pl.ds(base, B_PER_W)], idx_v)
        copies = []
        for j in range(CHUNKS_PER_W):
            copies.append(
                pltpu.async_copy(
                    table_hbm.at[idx_v.at[pl.ds(j * CHUNK, CHUNK)]],
                    rows_v.at[pl.ds(j * CHUNK, CHUNK)],
                    sem,
                )
            )
        for c in copies:
            c.wait()
        pltpu.sync_copy(rows_v, out_hbm.at[pl.ds(base, B_PER_W)])

    return _sc_gather


BLK = 2048
_GRID = N_ROWS // BLK


def _tc_body(pair_ref, par_ref, te_pair_ref, te_par_ref,
             fcw_ref, fcb_ref, fc2t_ref, fc2b_ref, out_ref):
    hi = lax.Precision.HIGHEST
    te = jnp.where(te_par_ref[...] > 0, te_pair_ref[:, D:], te_pair_ref[:, :D])
    attach = jnp.where(par_ref[...] > 0, pair_ref[:, D:], pair_ref[:, :D])
    tr = fcb_ref[...]
    for p in range(P):
        tr = tr + jnp.dot(te[p:p + 1, :], fcw_ref[p],
                          preferred_element_type=jnp.float32, precision=hi)
    c = jnp.dot(tr, fc2t_ref[...],
                preferred_element_type=jnp.float32, precision=hi) + fc2b_ref[...]
    out_ref[...] = c - jnp.dot(attach, fc2t_ref[...],
                               preferred_element_type=jnp.float32, precision=hi)


_tc_call = pl.pallas_call(
    _tc_body,
    grid=(_GRID,),
    in_specs=[
        pl.BlockSpec((BLK, 2 * D), lambda i: (i, 0)),
        pl.BlockSpec((BLK, 1), lambda i: (i, 0)),
        pl.BlockSpec((P, 2 * D), lambda i: (0, 0)),
        pl.BlockSpec((P, 1), lambda i: (0, 0)),
        pl.BlockSpec((P, D, D), lambda i: (0, 0, 0)),
        pl.BlockSpec((1, D), lambda i: (0, 0)),
        pl.BlockSpec((D, P + 1), lambda i: (0, 0)),
        pl.BlockSpec((1, P + 1), lambda i: (0, 0)),
    ],
    out_specs=pl.BlockSpec((BLK, P + 1), lambda i: (i, 0)),
    out_shape=jax.ShapeDtypeStruct((N_ROWS, P + 1), jnp.float32),
)


def kernel(term, attach_terms, device, table, fc_W, fc_b, fc2_W, fc2_b):
    del device
    idx = jnp.concatenate([
        attach_terms.astype(jnp.int32),
        term.astype(jnp.int32),
        jnp.zeros((B_TOTAL - N_ROWS - P,), jnp.int32),
    ])
    pair_idx = lax.shift_right_logical(idx, 1)
    parity = lax.bitwise_and(idx, 1).reshape(B_TOTAL, 1)
    g = _sc_gather_kernel()(table.reshape(NPAIR, 2 * D), pair_idx)
    fc_Wr = fc_W.reshape(D, P, D).transpose(1, 2, 0)
    return _tc_call(g[:N_ROWS], parity[:N_ROWS],
                    g[N_ROWS:N_ROWS + P], parity[N_ROWS:N_ROWS + P],
                    fc_Wr, fc_b.reshape(1, D), fc2_W.T, fc2_b.reshape(1, P + 1))


# speedup vs baseline: 8.6828x; 1.1110x over previous
import functools

import jax
import jax.numpy as jnp
from jax import lax
from jax.experimental import pallas as pl
from jax.experimental.pallas import tpu as pltpu
from jax.experimental.pallas import tpu_sc as plsc

N_ROWS = 16384
D = 64
P = 3
NC = 2
NS = 16
NW = NC * NS
CHUNK = 104
CHUNKS_PER_W = 5
B_PER_W = CHUNK * CHUNKS_PER_W
B_TOTAL = B_PER_W * NW
NPAIR = 500000


@functools.cache
def _sc_gather_kernel():
    mesh = plsc.VectorSubcoreMesh(core_axis_name="c", subcore_axis_name="s")

    @functools.partial(
        pl.kernel,
        mesh=mesh,
        out_type=jax.ShapeDtypeStruct((B_TOTAL, 2 * D), jnp.float32),
        scratch_types=[
            pltpu.VMEM((B_PER_W,), jnp.int32),
            pltpu.VMEM((B_PER_W, 2 * D), jnp.float32),
            pltpu.SemaphoreType.DMA,
        ],
    )
    def _sc_gather(table_hbm, idx_hbm, out_hbm, idx_v, rows_v, sem):
        wid = lax.axis_index("s") * NC + lax.axis_index("c")
        base = pl.multiple_of(wid * B_PER_W, 8)
        pltpu.sync_copy(idx_hbm.at[pl.ds(base, B_PER_W)], idx_v)
        copies = []
        for j in range(CHUNKS_PER_W):
            copies.append(
                pltpu.async_copy(
                    table_hbm.at[idx_v.at[pl.ds(j * CHUNK, CHUNK)]],
                    rows_v.at[pl.ds(j * CHUNK, CHUNK)],
                    sem,
                )
            )
        for c in copies:
            c.wait()
        pltpu.sync_copy(rows_v, out_hbm.at[pl.ds(base, B_PER_W)])

    return _sc_gather


BLK = 2048
_GRID = N_ROWS // BLK


def _tc_body(pair_ref, te_pair_ref,
             fcw_ref, fcb_ref, fc2t_ref, fc2b_ref, out_ref):
    hi = lax.Precision.HIGHEST
    te = te_pair_ref[:, :D]
    attach = pair_ref[:, :D]
    tr = fcb_ref[...]
    for p in range(P):
        tr = tr + jnp.dot(te[p:p + 1, :], fcw_ref[p],
                          preferred_element_type=jnp.float32, precision=hi)
    c = jnp.dot(tr, fc2t_ref[...],
                preferred_element_type=jnp.float32, precision=hi) + fc2b_ref[...]
    out_ref[...] = c - jnp.dot(attach, fc2t_ref[...],
                               preferred_element_type=jnp.float32, precision=hi)


_tc_call = pl.pallas_call(
    _tc_body,
    grid=(_GRID,),
    in_specs=[
        pl.BlockSpec((BLK, 2 * D), lambda i: (i, 0)),
        pl.BlockSpec((P, 2 * D), lambda i: (0, 0)),
        pl.BlockSpec((P, D, D), lambda i: (0, 0, 0)),
        pl.BlockSpec((1, D), lambda i: (0, 0)),
        pl.BlockSpec((D, P + 1), lambda i: (0, 0)),
        pl.BlockSpec((1, P + 1), lambda i: (0, 0)),
    ],
    out_specs=pl.BlockSpec((BLK, P + 1), lambda i: (i, 0)),
    out_shape=jax.ShapeDtypeStruct((N_ROWS, P + 1), jnp.float32),
)


def kernel(term, attach_terms, device, table, fc_W, fc_b, fc2_W, fc2_b):
    del device
    idx = jnp.concatenate([
        attach_terms.astype(jnp.int32),
        term.astype(jnp.int32),
        jnp.zeros((B_TOTAL - N_ROWS - P,), jnp.int32),
    ])
    table_pad = jnp.pad(table, ((0, 0), (0, D)))
    g = _sc_gather_kernel()(table_pad, idx)
    fc_Wr = fc_W.reshape(D, P, D).transpose(1, 2, 0)
    return _tc_call(g[:N_ROWS], g[N_ROWS:N_ROWS + P],
                    fc_Wr, fc_b.reshape(1, D), fc2_W.T, fc2_b.reshape(1, P + 1))
